# Initial kernel scaffold; baseline (speedup 1.0000x reference)
import jax
import jax.numpy as jnp
from jax.experimental import pallas as pl


def kernel(x, edge_index, edge_attr, batch, params):
    raise NotImplementedError("write your pallas kernel here")


# speedup vs baseline: 9.0288x; 9.0288x over previous
import functools

import jax
import jax.numpy as jnp
from jax.experimental import pallas as pl

H = 192
NA = 5


def _prep_body(u_ref, w1_ref, b1p_ref, rmask_ref, w2_ref, b2_ref, out_ref):
    a5 = jnp.dot(u_ref[...], w1_ref[...],
                 preferred_element_type=jnp.float32) + b1p_ref[...]
    m = jnp.dot(a5, w2_ref[...], preferred_element_type=jnp.float32)
    out_ref[...] = m + rmask_ref[...] * b2_ref[...]


def _prep_layer(upad, w1, b1pad, rowmask, w2, b2row):
    nb = 8
    bc = (H * H) // nb
    return pl.pallas_call(
        _prep_body,
        grid=(nb,),
        in_specs=[
            pl.BlockSpec((8, H), lambda j: (0, 0)),
            pl.BlockSpec((H, H), lambda j: (0, 0)),
            pl.BlockSpec((8, H), lambda j: (0, 0)),
            pl.BlockSpec((8, 1), lambda j: (0, 0)),
            pl.BlockSpec((H, bc), lambda j: (0, j)),
            pl.BlockSpec((1, bc), lambda j: (0, j)),
        ],
        out_specs=pl.BlockSpec((8, bc), lambda j: (0, j)),
        out_shape=jax.ShapeDtypeStruct((8, H * H), jnp.float32),
    )(upad, w1, b1pad, rowmask, w2, b2row)


def _main_body(x_ref, src_ref, dst_ref, ea5_ref, chebw_ref, chebb_ref,
               m3_ref, roots_ref, biases_ref, gammas_ref, betas_ref,
               linw_ref, linb_ref, out_ref):
    E = src_ref.shape[0]
    N = x_ref.shape[0]
    f32 = jnp.float32

    col = jax.lax.broadcasted_iota(jnp.int32, (E, N), 1)
    G = (src_ref[...] == col).astype(f32)
    S = (dst_ref[...] == col).astype(f32)

    deg = jnp.sum(G, axis=0).reshape(N, 1)
    dis = jnp.where(deg > 0, jax.lax.rsqrt(jnp.maximum(deg, 1e-12)), 0.0)
    dis_src = jnp.dot(G, dis, preferred_element_type=f32)
    dis_dst = jnp.dot(S, dis, preferred_element_type=f32)
    norm = -(dis_src * dis_dst)

    def lhat(y):
        t = norm * jnp.dot(G, y, preferred_element_type=f32)
        return jax.lax.dot_general(S, t, (((0,), (0,)), ((), ())),
                                   preferred_element_type=f32)

    tx0 = x_ref[...]
    tx1 = lhat(tx0)
    tx2 = 2.0 * lhat(tx1) - tx0
    tx3 = 2.0 * lhat(tx2) - tx1
    tx4 = 2.0 * lhat(tx3) - tx2
    txcat = jnp.concatenate([tx0, tx1, tx2, tx3, tx4], axis=1)
    xl = jnp.dot(txcat, chebw_ref[...],
                 preferred_element_type=f32) + chebb_ref[...]

    ea5 = ea5_ref[...]

    def nnconv(v, l):
        ycat = jnp.dot(v, m3_ref[l], preferred_element_type=f32)
        z = jnp.dot(G, ycat, preferred_element_type=f32)
        msg = ea5[:, 0:1] * z[:, 0:H]
        for a in range(1, NA):
            msg = msg + ea5[:, a:a + 1] * z[:, a * H:(a + 1) * H]
        agg = jax.lax.dot_general(S, msg, (((0,), (0,)), ((), ())),
                                  preferred_element_type=f32)
        return agg + jnp.dot(v, roots_ref[l],
                             preferred_element_type=f32) + biases_ref[l:l + 1, :]

    def ln_relu(v, l):
        mu = jnp.mean(v, axis=1, keepdims=True)
        var = jnp.mean((v - mu) ** 2, axis=1, keepdims=True)
        nrm = (v - mu) * jax.lax.rsqrt(var + 1e-5)
        return jnp.maximum(nrm * gammas_ref[l:l + 1, :] + betas_ref[l:l + 1, :],
                           0.0)

    xl = nnconv(xl, 0)
    for l in (1, 2, 3):
        xl = xl + nnconv(ln_relu(xl, l), l)
    h = ln_relu(xl, 0)
    out_ref[...] = jnp.dot(h, linw_ref[...],
                           preferred_element_type=f32) + linb_ref[...]


def kernel(x, edge_index, edge_attr, batch, params):
    N = x.shape[0]
    E = edge_index.shape[1]
    layers = params['layers']

    upad = jnp.zeros((8, H), jnp.float32)
    upad = upad.at[0:4, :].set(params['enc_W'])
    upad = upad.at[4, :].set(params['enc_b'])
    rowmask = jnp.zeros((8, 1), jnp.float32).at[4, 0].set(1.0)
    m3cats = []
    for l in layers:
        b1pad = jnp.zeros((8, H), jnp.float32).at[4, :].set(l['nn_b1'])
        m = _prep_layer(upad, l['nn_W1'], b1pad, rowmask,
                        l['nn_W2'], l['nn_b2'].reshape(1, H * H))
        m3cats.append(jnp.transpose(m[:NA].reshape(NA, H, H),
                                    (1, 0, 2)).reshape(H, NA * H))
    m3 = jnp.stack(m3cats)
    roots = jnp.stack([l['root'] for l in layers])
    biases = jnp.stack([l['bias'] for l in layers])
    gammas = jnp.stack([l['gamma'] for l in layers])
    betas = jnp.stack([l['beta'] for l in layers])

    src = edge_index[0].astype(jnp.int32).reshape(E, 1)
    dst = edge_index[1].astype(jnp.int32).reshape(E, 1)
    ea5 = jnp.concatenate([edge_attr, jnp.ones((E, 1), jnp.float32)], axis=1)
    chebw = params['cheb_W'].reshape(NA * 4, H)
    chebb = params['cheb_b'].reshape(1, H)
    linw = params['lin_W']
    linb = params['lin_b'].reshape(1, 2)

    return pl.pallas_call(
        _main_body,
        out_shape=jax.ShapeDtypeStruct((N, 2), jnp.float32),
    )(x, src, dst, ea5, chebw, chebb, m3, roots, biases, gammas, betas,
      linw, linb)
